# baseline (speedup 1.0000x reference)
import jax
import jax.numpy as jnp
from jax import lax
from jax.experimental import pallas as pl
from jax.experimental.pallas import tpu as pltpu
from jax.experimental.pallas import tpu_sc as plsc

NC, NS = 2, 16
NW = NC * NS
CHUNK = 128
POS_LEN = 200
NBUF = 6
PRE = 4
LOOK = 2


def _sc_body(idx_hbm, tok_hbm, pos2_hbm, out_hbm,
             idx_v, bufs, psem, gsem, ssem):
    wid = lax.axis_index("s") * NC + lax.axis_index("c")
    n_rows = idx_hbm.shape[0] // NW
    n_chunks = n_rows // CHUNK
    base = wid * n_chunks

    pltpu.sync_copy(idx_hbm.at[pl.ds(wid * n_rows, n_rows)], idx_v)

    def pos_slice(c):
        pos_base = pl.multiple_of(lax.rem(c * CHUNK, POS_LEN), 8)
        return pos2_hbm.at[pl.ds(pos_base, CHUNK)]

    def start_prefill(c):
        b = lax.rem(c, NBUF)
        pltpu.async_copy(pos_slice(c), bufs.at[b], psem.at[b])

    def wait_prefill(c):
        b = lax.rem(c, NBUF)
        pltpu.make_async_copy(pos_slice(c), bufs.at[b], psem.at[b]).wait()

    def chunk_idx(c):
        return idx_v.at[pl.ds(c * CHUNK, CHUNK)]

    def start_gather(c):
        b = lax.rem(c, NBUF)
        pltpu.async_copy(tok_hbm.at[chunk_idx(c)], bufs.at[b], gsem.at[b], add=True)

    def wait_gather(c):
        b = lax.rem(c, NBUF)
        pltpu.make_async_copy(tok_hbm.at[chunk_idx(c)], bufs.at[b], gsem.at[b]).wait()

    def start_scatter(c):
        b = lax.rem(c, NBUF)
        pltpu.async_copy(
            bufs.at[b], out_hbm.at[pl.ds((base + c) * CHUNK, CHUNK)], ssem.at[b])

    def wait_scatter(c):
        b = lax.rem(c, NBUF)
        pltpu.make_async_copy(
            bufs.at[b], out_hbm.at[pl.ds((base + c) * CHUNK, CHUNK)], ssem.at[b]).wait()

    for i in range(PRE):
        start_prefill(i)
    for i in range(LOOK):
        wait_prefill(i)
        start_gather(i)
    for i in range(LOOK):
        start_prefill(i + PRE)
        wait_prefill(i + LOOK)
        start_gather(i + LOOK)
        wait_gather(i)
        start_scatter(i)

    def steady(i, carry):
        wait_scatter(i - LOOK)
        start_prefill(i + PRE)
        wait_prefill(i + LOOK)
        start_gather(i + LOOK)
        wait_gather(i)
        start_scatter(i)
        return carry

    lax.fori_loop(LOOK, n_chunks - PRE, steady, 0)

    for i in range(n_chunks - PRE, n_chunks - LOOK):
        wait_scatter(i - LOOK)
        wait_prefill(i + LOOK)
        start_gather(i + LOOK)
        wait_gather(i)
        start_scatter(i)
    for i in range(n_chunks - LOOK, n_chunks):
        wait_scatter(i - LOOK)
        wait_gather(i)
        start_scatter(i)
    for i in range(n_chunks - LOOK, n_chunks):
        wait_scatter(i)


def kernel(inputs, token_table, pos_table):
    batch, seq_len = inputs.shape
    d_model = token_table.shape[1]
    total = batch * seq_len
    idx_flat = inputs.reshape(total).astype(jnp.int32)
    pos2 = jnp.concatenate([pos_table, pos_table[:CHUNK]], axis=0)
    n_rows = total // NW

    mesh = plsc.VectorSubcoreMesh(core_axis_name="c", subcore_axis_name="s")
    out = pl.kernel(
        _sc_body,
        out_type=jax.ShapeDtypeStruct((total, d_model), jnp.float32),
        mesh=mesh,
        scratch_types=[
            pltpu.VMEM((n_rows,), jnp.int32),
            pltpu.VMEM((NBUF, CHUNK, d_model), jnp.float32),
            pltpu.SemaphoreType.DMA((NBUF,)),
            pltpu.SemaphoreType.DMA((NBUF,)),
            pltpu.SemaphoreType.DMA((NBUF,)),
        ],
    )(idx_flat, token_table, pos2)
    return out.reshape(batch, seq_len, d_model)


# speedup vs baseline: 2.5526x; 2.5526x over previous
import jax
import jax.numpy as jnp
from jax import lax
from jax.experimental import pallas as pl
from jax.experimental.pallas import tpu as pltpu
from jax.experimental.pallas import tpu_sc as plsc

NC, NS, LANES = 2, 16, 16
NW = NC * NS
CHUNK = 128
POS_LEN = 200
NBUF = 7
PRE = 5
LOOK = 2


def _sc_body(idx_hbm, tok_hbm, pos_hbm, out_hbm,
             idx_v, pos_sh, bufs, isem, psem, gsem, ssem):
    wid = lax.axis_index("s") * NC + lax.axis_index("c")
    n_rows = idx_hbm.shape[0] // NW
    n_chunks = n_rows // CHUNK
    base = wid * n_chunks

    idx_cp = pltpu.async_copy(idx_hbm.at[pl.ds(wid * n_rows, n_rows)], idx_v, isem)

    @pl.when(lax.axis_index("s") == 0)
    def _():
        pltpu.sync_copy(pos_hbm, pos_sh.at[pl.ds(0, POS_LEN)])
        pltpu.sync_copy(pos_hbm.at[pl.ds(0, CHUNK)],
                        pos_sh.at[pl.ds(POS_LEN, CHUNK)])

    plsc.subcore_barrier()

    def pos_slice(c):
        pos_base = pl.multiple_of(lax.rem(c * CHUNK, POS_LEN), 8)
        return pos_sh.at[pl.ds(pos_base, CHUNK)]

    def start_prefill(c):
        b = lax.rem(c, NBUF)
        pltpu.async_copy(pos_slice(c), bufs.at[b], psem.at[b])

    def wait_prefill(c):
        b = lax.rem(c, NBUF)
        pltpu.make_async_copy(pos_slice(c), bufs.at[b], psem.at[b]).wait()

    def chunk_idx(c):
        return idx_v.at[pl.ds(c * CHUNK, CHUNK)]

    def start_gather(c):
        b = lax.rem(c, NBUF)
        pltpu.async_copy(tok_hbm.at[chunk_idx(c)], bufs.at[b], gsem.at[b], add=True)

    def wait_gather(c):
        b = lax.rem(c, NBUF)
        pltpu.make_async_copy(tok_hbm.at[chunk_idx(c)], bufs.at[b], gsem.at[b]).wait()

    def start_scatter(c):
        b = lax.rem(c, NBUF)
        pltpu.async_copy(
            bufs.at[b], out_hbm.at[pl.ds((base + c) * CHUNK, CHUNK)], ssem.at[b])

    def wait_scatter(c):
        b = lax.rem(c, NBUF)
        pltpu.make_async_copy(
            bufs.at[b], out_hbm.at[pl.ds((base + c) * CHUNK, CHUNK)], ssem.at[b]).wait()

    for i in range(PRE):
        start_prefill(i)
    idx_cp.wait()
    for i in range(LOOK):
        wait_prefill(i)
        start_gather(i)
    for i in range(LOOK):
        start_prefill(i + PRE)
        wait_prefill(i + LOOK)
        start_gather(i + LOOK)
        wait_gather(i)
        start_scatter(i)

    def steady(i, carry):
        wait_scatter(i - LOOK)
        start_prefill(i + PRE)
        wait_prefill(i + LOOK)
        start_gather(i + LOOK)
        wait_gather(i)
        start_scatter(i)
        return carry

    lax.fori_loop(LOOK, n_chunks - PRE, steady, 0)

    for i in range(n_chunks - PRE, n_chunks - LOOK):
        wait_scatter(i - LOOK)
        wait_prefill(i + LOOK)
        start_gather(i + LOOK)
        wait_gather(i)
        start_scatter(i)
    for i in range(n_chunks - LOOK, n_chunks):
        wait_scatter(i - LOOK)
        wait_gather(i)
        start_scatter(i)
    for i in range(n_chunks - LOOK, n_chunks):
        wait_scatter(i)


def kernel(inputs, token_table, pos_table):
    batch, seq_len = inputs.shape
    d_model = token_table.shape[1]
    total = batch * seq_len
    idx_flat = inputs.reshape(total).astype(jnp.int32)
    n_rows = total // NW

    mesh = plsc.VectorSubcoreMesh(core_axis_name="c", subcore_axis_name="s")
    out = pl.kernel(
        _sc_body,
        out_type=jax.ShapeDtypeStruct((total, d_model), jnp.float32),
        mesh=mesh,
        scratch_types=[
            pltpu.VMEM((n_rows,), jnp.int32),
            pltpu.VMEM_SHARED((seq_len + CHUNK, d_model), jnp.float32),
            pltpu.VMEM((NBUF, CHUNK, d_model), jnp.float32),
            pltpu.SemaphoreType.DMA,
            pltpu.SemaphoreType.DMA((NBUF,)),
            pltpu.SemaphoreType.DMA((NBUF,)),
            pltpu.SemaphoreType.DMA((NBUF,)),
        ],
    )(idx_flat, token_table, pos_table)
    return out.reshape(batch, seq_len, d_model)


# speedup vs baseline: 2.5664x; 1.0054x over previous
import jax
import jax.numpy as jnp
from jax import lax
from jax.experimental import pallas as pl
from jax.experimental.pallas import tpu as pltpu
from jax.experimental.pallas import tpu_sc as plsc

NC, NS, LANES = 2, 16, 16
NW = NC * NS
CHUNK = 128
POS_LEN = 200
NBUF = 7
PRE = 4
LOOK = 2
SWAIT = NBUF - PRE


def _sc_body(idx_hbm, tok_hbm, pos_hbm, out_hbm,
             idx_v, pos_sh, bufs, isem, psem, gsem, ssem):
    wid = lax.axis_index("s") * NC + lax.axis_index("c")
    n_rows = idx_hbm.shape[0] // NW
    n_chunks = n_rows // CHUNK
    base = wid * n_chunks

    idx_cp = pltpu.async_copy(idx_hbm.at[pl.ds(wid * n_rows, n_rows)], idx_v, isem)

    @pl.when(lax.axis_index("s") == 0)
    def _():
        pltpu.sync_copy(pos_hbm, pos_sh.at[pl.ds(0, POS_LEN)])
        pltpu.sync_copy(pos_hbm.at[pl.ds(0, CHUNK)],
                        pos_sh.at[pl.ds(POS_LEN, CHUNK)])

    plsc.subcore_barrier()

    def pos_slice(c):
        pos_base = pl.multiple_of(lax.rem(c * CHUNK, POS_LEN), 8)
        return pos_sh.at[pl.ds(pos_base, CHUNK)]

    def start_prefill(c):
        b = lax.rem(c, NBUF)
        pltpu.async_copy(pos_slice(c), bufs.at[b], psem.at[b])

    def wait_prefill(c):
        b = lax.rem(c, NBUF)
        pltpu.make_async_copy(pos_slice(c), bufs.at[b], psem.at[b]).wait()

    def chunk_idx(c):
        return idx_v.at[pl.ds(c * CHUNK, CHUNK)]

    def start_gather(c):
        b = lax.rem(c, NBUF)
        pltpu.async_copy(tok_hbm.at[chunk_idx(c)], bufs.at[b], gsem.at[b], add=True)

    def wait_gather(c):
        b = lax.rem(c, NBUF)
        pltpu.make_async_copy(tok_hbm.at[chunk_idx(c)], bufs.at[b], gsem.at[b]).wait()

    def start_scatter(c):
        b = lax.rem(c, NBUF)
        pltpu.async_copy(
            bufs.at[b], out_hbm.at[pl.ds((base + c) * CHUNK, CHUNK)], ssem.at[b])

    def wait_scatter(c):
        b = lax.rem(c, NBUF)
        pltpu.make_async_copy(
            bufs.at[b], out_hbm.at[pl.ds((base + c) * CHUNK, CHUNK)], ssem.at[b]).wait()

    for i in range(PRE):
        start_prefill(i)
    idx_cp.wait()
    for i in range(LOOK):
        wait_prefill(i)
        start_gather(i)

    def step(i, carry):
        @pl.when(i >= SWAIT)
        def _():
            wait_scatter(i - SWAIT)

        @pl.when(i + PRE < n_chunks)
        def _():
            start_prefill(i + PRE)

        @pl.when(i + LOOK < n_chunks)
        def _():
            wait_prefill(i + LOOK)
            start_gather(i + LOOK)

        wait_gather(i)
        start_scatter(i)
        return carry

    lax.fori_loop(0, n_chunks, step, 0)

    for i in range(n_chunks - SWAIT, n_chunks):
        wait_scatter(i)


def kernel(inputs, token_table, pos_table):
    batch, seq_len = inputs.shape
    d_model = token_table.shape[1]
    total = batch * seq_len
    idx_flat = inputs.reshape(total).astype(jnp.int32)
    n_rows = total // NW

    mesh = plsc.VectorSubcoreMesh(core_axis_name="c", subcore_axis_name="s")
    out = pl.kernel(
        _sc_body,
        out_type=jax.ShapeDtypeStruct((total, d_model), jnp.float32),
        mesh=mesh,
        scratch_types=[
            pltpu.VMEM((n_rows,), jnp.int32),
            pltpu.VMEM_SHARED((seq_len + CHUNK, d_model), jnp.float32),
            pltpu.VMEM((NBUF, CHUNK, d_model), jnp.float32),
            pltpu.SemaphoreType.DMA,
            pltpu.SemaphoreType.DMA((NBUF,)),
            pltpu.SemaphoreType.DMA((NBUF,)),
            pltpu.SemaphoreType.DMA((NBUF,)),
        ],
    )(idx_flat, token_table, pos_table)
    return out.reshape(batch, seq_len, d_model)


# speedup vs baseline: 2.6000x; 1.0131x over previous
import jax
import jax.numpy as jnp
from jax import lax
from jax.experimental import pallas as pl
from jax.experimental.pallas import tpu as pltpu
from jax.experimental.pallas import tpu_sc as plsc

NC, NS = 2, 16
NW = NC * NS
CHUNK = 128
POS_LEN = 200
NBUF = 7
PRE = 4
LOOK = 3
SWAIT = NBUF - PRE


def _sc_body(idx_hbm, tok_hbm, pos_hbm, out_hbm,
             idx_v, pos_sh, bufs, isem, psem, gsem, ssem):
    wid = lax.axis_index("s") * NC + lax.axis_index("c")
    n_rows = idx_hbm.shape[0] // NW
    n_chunks = n_rows // CHUNK
    base = wid * n_chunks

    idx_cp = pltpu.async_copy(idx_hbm.at[pl.ds(wid * n_rows, n_rows)], idx_v, isem)

    @pl.when(lax.axis_index("s") == 0)
    def _():
        pltpu.sync_copy(pos_hbm, pos_sh.at[pl.ds(0, POS_LEN)])
        pltpu.sync_copy(pos_hbm.at[pl.ds(0, CHUNK)],
                        pos_sh.at[pl.ds(POS_LEN, CHUNK)])

    plsc.subcore_barrier()

    def pos_slice(c):
        pos_base = pl.multiple_of(lax.rem(c * CHUNK, POS_LEN), 8)
        return pos_sh.at[pl.ds(pos_base, CHUNK)]

    def start_prefill(c):
        b = lax.rem(c, NBUF)
        pltpu.async_copy(pos_slice(c), bufs.at[b], psem.at[b])

    def wait_prefill(c):
        b = lax.rem(c, NBUF)
        pltpu.make_async_copy(pos_slice(c), bufs.at[b], psem.at[b]).wait()

    HALF = CHUNK // 2

    def half_idx(c, h):
        return idx_v.at[pl.ds(c * CHUNK + h * HALF, HALF)]

    def half_buf(c, h):
        b = lax.rem(c, NBUF)
        return bufs.at[b, pl.ds(h * HALF, HALF)]

    def start_gather(c):
        b = lax.rem(c, NBUF)
        for h in range(2):
            pltpu.async_copy(tok_hbm.at[half_idx(c, h)], half_buf(c, h),
                             gsem.at[b, h], add=True)

    def wait_gather(c):
        b = lax.rem(c, NBUF)
        for h in range(2):
            pltpu.make_async_copy(tok_hbm.at[half_idx(c, h)], half_buf(c, h),
                                  gsem.at[b, h]).wait()

    def start_scatter(c):
        b = lax.rem(c, NBUF)
        pltpu.async_copy(
            bufs.at[b], out_hbm.at[pl.ds((base + c) * CHUNK, CHUNK)], ssem.at[b])

    def wait_scatter(c):
        b = lax.rem(c, NBUF)
        pltpu.make_async_copy(
            bufs.at[b], out_hbm.at[pl.ds((base + c) * CHUNK, CHUNK)], ssem.at[b]).wait()

    for i in range(PRE):
        start_prefill(i)
    idx_cp.wait()
    for i in range(LOOK):
        wait_prefill(i)
        start_gather(i)

    def step(i, carry):
        @pl.when(i >= SWAIT)
        def _():
            wait_scatter(i - SWAIT)

        @pl.when(i + PRE < n_chunks)
        def _():
            start_prefill(i + PRE)

        @pl.when(i + LOOK < n_chunks)
        def _():
            wait_prefill(i + LOOK)
            start_gather(i + LOOK)

        wait_gather(i)
        start_scatter(i)
        return carry

    lax.fori_loop(0, n_chunks, step, 0)

    for i in range(n_chunks - SWAIT, n_chunks):
        wait_scatter(i)


def kernel(inputs, token_table, pos_table):
    batch, seq_len = inputs.shape
    d_model = token_table.shape[1]
    total = batch * seq_len
    idx_flat = inputs.reshape(total).astype(jnp.int32)
    n_rows = total // NW

    mesh = plsc.VectorSubcoreMesh(core_axis_name="c", subcore_axis_name="s")
    out = pl.kernel(
        _sc_body,
        out_type=jax.ShapeDtypeStruct((total, d_model), jnp.float32),
        mesh=mesh,
        scratch_types=[
            pltpu.VMEM((n_rows,), jnp.int32),
            pltpu.VMEM_SHARED((seq_len + CHUNK, d_model), jnp.float32),
            pltpu.VMEM((NBUF, CHUNK, d_model), jnp.float32),
            pltpu.SemaphoreType.DMA,
            pltpu.SemaphoreType.DMA((NBUF,)),
            pltpu.SemaphoreType.DMA((NBUF, 2)),
            pltpu.SemaphoreType.DMA((NBUF,)),
        ],
    )(idx_flat, token_table, pos_table)
    return out.reshape(batch, seq_len, d_model)
